# Initial kernel scaffold; baseline (speedup 1.0000x reference)
import jax
import jax.numpy as jnp
from jax.experimental import pallas as pl


def kernel(nlist, extended_coord, extended_atype, mean, stddev, W0, b0, W1, b1, W2, b2):
    raise NotImplementedError("write your pallas kernel here")


# speedup vs baseline: 12.9189x; 12.9189x over previous
import functools

import jax
import jax.numpy as jnp
from jax import lax
from jax.experimental import pallas as pl
from jax.experimental.pallas import tpu as pltpu
from jax.experimental.pallas import tpu_sc as plsc

_NB, _NLOC, _NALL, _NNEI, _NG = 2, 1024, 1280, 60, 8
_RMIN, _RMAX = 0.5, 4.0
_NW = 32
_APT = (_NB * _NLOC) // _NW
_NR = 64
_SPT = _NR * _APT
_SENT = _NB * _NALL
_FAR = 1.0e4
_TILE = 128
_GRID = (_NB * _NLOC) // _TILE
_BLOCKS = ((0, True, True, 1.0 / 400.0),
           (2, True, False, 1.0 / 800.0),
           (3, False, False, 1.0 / 1600.0))


def _sc_gather(idx_chunks, cx, cy, cz):
    mesh = plsc.VectorSubcoreMesh(core_axis_name="c", subcore_axis_name="s",
                                  num_cores=2, num_subcores=16)

    @functools.partial(
        pl.kernel,
        out_type=jax.ShapeDtypeStruct((_NW * 3 * _SPT,), jnp.float32),
        mesh=mesh,
        scratch_types=[
            pltpu.VMEM((_SPT,), jnp.int32),
            pltpu.VMEM((_SPT,), jnp.float32),
            pltpu.VMEM((_SPT,), jnp.float32),
            pltpu.VMEM((_SPT,), jnp.float32),
            pltpu.VMEM((_APT,), jnp.float32),
            pltpu.VMEM((_APT,), jnp.float32),
            pltpu.VMEM((_APT,), jnp.float32),
            pltpu.SemaphoreType.DMA,
        ],
    )
    def run(idx_hbm, cx_hbm, cy_hbm, cz_hbm, out_hbm,
            idx_v, sx_v, sy_v, sz_v, hx_v, hy_v, hz_v, sem):
        wid = lax.axis_index("s") * 2 + lax.axis_index("c")
        pltpu.sync_copy(idx_hbm.at[pl.ds(wid * _SPT, _SPT)], idx_v)
        tiles_per_batch = _NW // _NB
        batch = wid // tiles_per_batch
        a0g = batch * _NALL + (wid % tiles_per_batch) * _APT
        tabs = (cx_hbm, cy_hbm, cz_hbm)
        stage = (sx_v, sy_v, sz_v)
        home = (hx_v, hy_v, hz_v)
        for comp in range(3):
            pltpu.sync_copy(tabs[comp].at[pl.ds(a0g, _APT)], home[comp])
            pltpu.async_copy(tabs[comp].at[idx_v], stage[comp], sem).wait()

        def body(k, carry):
            off = k * 16
            hoff = lax.rem(k, _APT // 16) * 16
            for comp in range(3):
                d = stage[comp][pl.ds(off, 16)] - home[comp][pl.ds(hoff, 16)]
                stage[comp][pl.ds(off, 16)] = d
            return carry

        lax.fori_loop(0, _SPT // 16, body, 0)
        for comp in range(3):
            pltpu.sync_copy(
                stage[comp],
                out_hbm.at[pl.ds((wid * 3 + comp) * _SPT, _SPT)])

    return run(idx_chunks, cx, cy, cz)


def _tc_body(atype_ref, diff_ref, tav_ref, tsd_ref,
             w0_ref, b0_ref, w1_ref, b1_ref, w2_ref, b2_ref, out_ref):
    sel0 = jnp.broadcast_to((atype_ref[0, 0, :] == 0)[None, :], (_NR, _TILE))
    d = [jnp.concatenate([diff_ref[0, comp], diff_ref[1, comp]], axis=1)
         for comp in range(3)]
    len2 = d[0] * d[0] + d[1] * d[1] + d[2] * d[2]
    dist = jnp.sqrt(len2)
    uu = (dist - _RMIN) * (1.0 / (_RMAX - _RMIN))
    vv = uu * uu * uu * (-6.0 * uu * uu + 15.0 * uu - 10.0) + 1.0
    w = jnp.where(dist >= _RMAX, 0.0, jnp.where(dist <= _RMIN, 1.0, vv))
    ll = dist * dist
    rr = []
    for comp in range(3):
        ta = jnp.where(sel0, tav_ref[0, comp], tav_ref[1, comp])
        td = jnp.where(sel0, tsd_ref[0, comp], tsd_ref[1, comp])
        rr.append(((d[comp] / ll) * w - ta) / td)
    ra = [r[0:24] for r in rr]
    rb = [r[24:_NR] for r in rr]

    acc = [jnp.zeros((_TILE,), jnp.float32) for _ in range(_NG)]
    for e, j_is_a, k_is_a, scale in _BLOCKS:
        xj = ra if j_is_a else rb
        yk = ra if k_is_a else rb
        nj, nk = xj[0].shape[0], yk[0].shape[0]
        env = None
        for comp in range(3):
            aa = jnp.broadcast_to(xj[comp][:, None, :], (nj, nk, _TILE))
            bb = jnp.broadcast_to(yk[comp][None, :, :], (nj, nk, _TILE))
            t = (aa * bb).reshape(nj * nk, _TILE)
            env = t if env is None else env + t
        h1 = [jnp.tanh(env * w0_ref[e * 2 + c] + b0_ref[e * 2 + c]) + env
              for c in range(2)]
        h2 = []
        for dd in range(4):
            z = (h1[0] * w1_ref[e * 8 + dd] + h1[1] * w1_ref[e * 8 + 4 + dd]
                 + b1_ref[e * 4 + dd])
            h2.append(jnp.tanh(z) + h1[dd % 2])
        envs = env * scale
        for mm in range(_NG):
            z = (h2[0] * w2_ref[e * 32 + mm] + h2[1] * w2_ref[e * 32 + 8 + mm]
                 + h2[2] * w2_ref[e * 32 + 16 + mm]
                 + h2[3] * w2_ref[e * 32 + 24 + mm] + b2_ref[e * 8 + mm])
            g = jnp.tanh(z) + h2[mm % 4]
            acc[mm] = acc[mm] + jnp.sum(envs * g, axis=0)
    out_ref[...] = jnp.stack(acc, axis=0)


def _tc_call(atype, diff, tav, tsd, params):
    smem = pl.BlockSpec(memory_space=pltpu.SMEM)
    return pl.pallas_call(
        _tc_body,
        grid=(_GRID,),
        in_specs=[
            pl.BlockSpec((1, 1, _TILE), lambda i: (i, 0, 0)),
            pl.BlockSpec((2, 3, _NR, _APT), lambda i: (i, 0, 0, 0)),
            pl.BlockSpec((2, 3, _NR, _TILE), lambda i: (0, 0, 0, 0)),
            pl.BlockSpec((2, 3, _NR, _TILE), lambda i: (0, 0, 0, 0)),
            smem, smem, smem, smem, smem, smem,
        ],
        out_specs=pl.BlockSpec((_NG, _TILE), lambda i: (0, i)),
        out_shape=jax.ShapeDtypeStruct((_NG, _NB * _NLOC), jnp.float32),
    )(atype, diff, tav, tsd, *params)


def _pad_cols(x, padval):
    cols = x.astype(jnp.float32)[:, :, 1:4].transpose(0, 2, 1)
    padded = jnp.concatenate(
        [cols[:, :, :20], jnp.full((2, 3, 4), padval, jnp.float32),
         cols[:, :, 20:]], axis=2)
    return jnp.broadcast_to(padded[..., None], (2, 3, _NR, _TILE))


def _prep_indices(nlist):
    off = (jnp.arange(_NB, dtype=jnp.int32) * _NALL)[:, None, None]
    idx = jnp.where(nlist >= 0, nlist + off, _SENT)
    idx = idx.reshape(_NB * _NLOC, _NNEI)
    pad = jnp.full((_NB * _NLOC, 4), _SENT, jnp.int32)
    idx = jnp.concatenate([idx[:, :20], pad, idx[:, 20:]], axis=1)
    return idx.reshape(_NW, _APT, _NR).transpose(0, 2, 1).reshape(_NW, _SPT)


def kernel(nlist, extended_coord, extended_atype, mean, stddev,
           W0, b0, W1, b1, W2, b2):
    nlist = nlist.astype(jnp.int32)
    coord = extended_coord.astype(jnp.float32).reshape(_NB * _NALL, 3)
    sent = jnp.array([[_FAR, 0.0, 0.0]], jnp.float32)
    tables = jnp.concatenate([coord, sent], axis=0).T
    atype = extended_atype.astype(jnp.int32)[:, :_NLOC].reshape(
        _GRID, 1, _TILE)
    idx_chunks = _prep_indices(nlist).reshape(-1)
    diff = _sc_gather(idx_chunks, tables[0], tables[1], tables[2])
    diff = diff.reshape(_NW, 3, _NR, _APT)
    tav = _pad_cols(mean, 0.0)
    tsd = _pad_cols(stddev, 1.0)
    params = [p.astype(jnp.float32).reshape(-1)
              for p in (W0, b0, W1, b1, W2, b2)]
    outt = _tc_call(atype, diff, tav, tsd, params)
    return outt.T.reshape(_NB, _NLOC, _NG)


# speedup vs baseline: 13.2339x; 1.0244x over previous
import functools

import jax
import jax.numpy as jnp
from jax import lax
from jax.experimental import pallas as pl
from jax.experimental.pallas import tpu as pltpu
from jax.experimental.pallas import tpu_sc as plsc

_NB, _NLOC, _NALL, _NNEI, _NG = 2, 1024, 1280, 60, 8
_RMIN, _RMAX = 0.5, 4.0
_NW = 32
_APT = (_NB * _NLOC) // _NW
_NR = 64
_SPT = _NR * _APT
_SENT = _NB * _NALL
_FAR = 1.0e4
_TILE = 128
_GRID = (_NB * _NLOC) // _TILE
_BLOCKS = ((0, True, True, 1.0 / 400.0),
           (2, True, False, 1.0 / 800.0),
           (3, False, False, 1.0 / 1600.0))


def _sc_gather(idx_chunks, cx, cy, cz):
    mesh = plsc.VectorSubcoreMesh(core_axis_name="c", subcore_axis_name="s",
                                  num_cores=2, num_subcores=16)

    @functools.partial(
        pl.kernel,
        out_type=jax.ShapeDtypeStruct((_NW * 3 * _SPT,), jnp.float32),
        mesh=mesh,
        scratch_types=[
            pltpu.VMEM((_SPT,), jnp.int32),
            pltpu.VMEM((_SPT,), jnp.float32),
            pltpu.VMEM((_SPT,), jnp.float32),
            pltpu.VMEM((_SPT,), jnp.float32),
            pltpu.VMEM((_APT,), jnp.float32),
            pltpu.VMEM((_APT,), jnp.float32),
            pltpu.VMEM((_APT,), jnp.float32),
            pltpu.SemaphoreType.DMA,
        ],
    )
    def run(idx_hbm, cx_hbm, cy_hbm, cz_hbm, out_hbm,
            idx_v, sx_v, sy_v, sz_v, hx_v, hy_v, hz_v, sem):
        wid = lax.axis_index("s") * 2 + lax.axis_index("c")
        pltpu.sync_copy(idx_hbm.at[pl.ds(wid * _SPT, _SPT)], idx_v)
        tiles_per_batch = _NW // _NB
        batch = wid // tiles_per_batch
        a0g = batch * _NALL + (wid % tiles_per_batch) * _APT
        tabs = (cx_hbm, cy_hbm, cz_hbm)
        stage = (sx_v, sy_v, sz_v)
        home = (hx_v, hy_v, hz_v)
        descs = [pltpu.async_copy(tabs[comp].at[idx_v], stage[comp], sem)
                 for comp in range(3)]
        for comp in range(3):
            pltpu.sync_copy(tabs[comp].at[pl.ds(a0g, _APT)], home[comp])
        for desc in descs:
            desc.wait()

        hregs = [[home[comp][pl.ds(u * 16, 16)] for u in range(_APT // 16)]
                 for comp in range(3)]

        def body(k, carry):
            for u in range(_APT // 16):
                off = k * _APT + u * 16
                for comp in range(3):
                    d = stage[comp][pl.ds(off, 16)] - hregs[comp][u]
                    stage[comp][pl.ds(off, 16)] = d
            return carry

        lax.fori_loop(0, _NR, body, 0)
        for comp in range(3):
            pltpu.sync_copy(
                stage[comp],
                out_hbm.at[pl.ds((wid * 3 + comp) * _SPT, _SPT)])

    return run(idx_chunks, cx, cy, cz)


def _tc_body(atype_ref, diff_ref, tav_ref, tsd_ref,
             w0_ref, b0_ref, w1_ref, b1_ref, w2_ref, b2_ref, out_ref):
    sel0 = jnp.broadcast_to((atype_ref[0, 0, :] == 0)[None, :], (_NR, _TILE))
    d = [jnp.concatenate([diff_ref[0, comp], diff_ref[1, comp]], axis=1)
         for comp in range(3)]
    len2 = d[0] * d[0] + d[1] * d[1] + d[2] * d[2]
    dist = jnp.sqrt(len2)
    uu = (dist - _RMIN) * (1.0 / (_RMAX - _RMIN))
    vv = uu * uu * uu * (-6.0 * uu * uu + 15.0 * uu - 10.0) + 1.0
    w = jnp.where(dist >= _RMAX, 0.0, jnp.where(dist <= _RMIN, 1.0, vv))
    ll = dist * dist
    rr = []
    for comp in range(3):
        ta = jnp.where(sel0, tav_ref[0, comp], tav_ref[1, comp])
        td = jnp.where(sel0, tsd_ref[0, comp], tsd_ref[1, comp])
        rr.append(((d[comp] / ll) * w - ta) / td)
    ra = [r[0:24] for r in rr]
    rb = [r[24:_NR] for r in rr]

    acc = [jnp.zeros((_TILE,), jnp.float32) for _ in range(_NG)]
    for e, j_is_a, k_is_a, scale in _BLOCKS:
        xj = ra if j_is_a else rb
        yk = ra if k_is_a else rb
        nj, nk = xj[0].shape[0], yk[0].shape[0]
        env = None
        for comp in range(3):
            aa = jnp.broadcast_to(xj[comp][:, None, :], (nj, nk, _TILE))
            bb = jnp.broadcast_to(yk[comp][None, :, :], (nj, nk, _TILE))
            t = (aa * bb).reshape(nj * nk, _TILE)
            env = t if env is None else env + t
        h1 = [jnp.tanh(env * w0_ref[e * 2 + c] + b0_ref[e * 2 + c]) + env
              for c in range(2)]
        h2 = []
        for dd in range(4):
            z = (h1[0] * w1_ref[e * 8 + dd] + h1[1] * w1_ref[e * 8 + 4 + dd]
                 + b1_ref[e * 4 + dd])
            h2.append(jnp.tanh(z) + h1[dd % 2])
        envs = env * scale
        for mm in range(_NG):
            z = (h2[0] * w2_ref[e * 32 + mm] + h2[1] * w2_ref[e * 32 + 8 + mm]
                 + h2[2] * w2_ref[e * 32 + 16 + mm]
                 + h2[3] * w2_ref[e * 32 + 24 + mm] + b2_ref[e * 8 + mm])
            g = jnp.tanh(z) + h2[mm % 4]
            acc[mm] = acc[mm] + jnp.sum(envs * g, axis=0)
    out_ref[...] = jnp.stack(acc, axis=0)


def _tc_call(atype, diff, tav, tsd, params):
    smem = pl.BlockSpec(memory_space=pltpu.SMEM)
    return pl.pallas_call(
        _tc_body,
        grid=(_GRID,),
        in_specs=[
            pl.BlockSpec((1, 1, _TILE), lambda i: (i, 0, 0)),
            pl.BlockSpec((2, 3, _NR, _APT), lambda i: (i, 0, 0, 0)),
            pl.BlockSpec((2, 3, _NR, _TILE), lambda i: (0, 0, 0, 0)),
            pl.BlockSpec((2, 3, _NR, _TILE), lambda i: (0, 0, 0, 0)),
            smem, smem, smem, smem, smem, smem,
        ],
        out_specs=pl.BlockSpec((_NG, _TILE), lambda i: (0, i)),
        out_shape=jax.ShapeDtypeStruct((_NG, _NB * _NLOC), jnp.float32),
    )(atype, diff, tav, tsd, *params)


def _pad_cols(x, padval):
    cols = x.astype(jnp.float32)[:, :, 1:4].transpose(0, 2, 1)
    padded = jnp.concatenate(
        [cols[:, :, :20], jnp.full((2, 3, 4), padval, jnp.float32),
         cols[:, :, 20:]], axis=2)
    return jnp.broadcast_to(padded[..., None], (2, 3, _NR, _TILE))


def _prep_indices(nlist):
    off = (jnp.arange(_NB, dtype=jnp.int32) * _NALL)[:, None, None]
    idx = jnp.where(nlist >= 0, nlist + off, _SENT)
    idx = idx.reshape(_NB * _NLOC, _NNEI)
    pad = jnp.full((_NB * _NLOC, 4), _SENT, jnp.int32)
    idx = jnp.concatenate([idx[:, :20], pad, idx[:, 20:]], axis=1)
    return idx.reshape(_NW, _APT, _NR).transpose(0, 2, 1).reshape(_NW, _SPT)


def kernel(nlist, extended_coord, extended_atype, mean, stddev,
           W0, b0, W1, b1, W2, b2):
    nlist = nlist.astype(jnp.int32)
    coord = extended_coord.astype(jnp.float32).reshape(_NB * _NALL, 3)
    sent = jnp.array([[_FAR, 0.0, 0.0]], jnp.float32)
    tables = jnp.concatenate([coord, sent], axis=0).T
    atype = extended_atype.astype(jnp.int32)[:, :_NLOC].reshape(
        _GRID, 1, _TILE)
    idx_chunks = _prep_indices(nlist).reshape(-1)
    diff = _sc_gather(idx_chunks, tables[0], tables[1], tables[2])
    diff = diff.reshape(_NW, 3, _NR, _APT)
    tav = _pad_cols(mean, 0.0)
    tsd = _pad_cols(stddev, 1.0)
    params = [p.astype(jnp.float32).reshape(-1)
              for p in (W0, b0, W1, b1, W2, b2)]
    outt = _tc_call(atype, diff, tav, tsd, params)
    return outt.T.reshape(_NB, _NLOC, _NG)


# speedup vs baseline: 18.4737x; 1.3959x over previous
import functools

import jax
import jax.numpy as jnp
from jax import lax
from jax.experimental import pallas as pl
from jax.experimental.pallas import tpu as pltpu
from jax.experimental.pallas import tpu_sc as plsc

_NB, _NLOC, _NALL, _NNEI, _NG = 2, 1024, 1280, 60, 8
_RMIN, _RMAX = 0.5, 4.0
_NW = 32
_APT = (_NB * _NLOC) // _NW
_NR = 64
_SPT = _NR * _APT
_SENT = _NB * _NALL
_NTAB = _SENT + 8
_FAR = 1.0e4
_TILE = 128
_GRID = (_NB * _NLOC) // _TILE
_BLOCKS = ((0, True, True, 1.0 / 400.0),
           (2, True, False, 1.0 / 800.0),
           (3, False, False, 1.0 / 1600.0))


def _sc_gather(idx_chunks, ctab):
    mesh = plsc.VectorSubcoreMesh(core_axis_name="c", subcore_axis_name="s",
                                  num_cores=2, num_subcores=16)


    @functools.partial(
        pl.kernel,
        out_type=jax.ShapeDtypeStruct((_NW * 3 * _SPT,), jnp.float32),
        mesh=mesh,
        scratch_types=[
            pltpu.VMEM((3 * _SPT,), jnp.int32),
            pltpu.VMEM((3 * _SPT,), jnp.float32),
            pltpu.VMEM((3 * _APT,), jnp.float32),
            pltpu.VMEM_SHARED((3 * _NTAB,), jnp.float32),
            pltpu.SemaphoreType.DMA,
        ],
    )
    def run(idx_hbm, ctab_hbm, out_hbm, idx_v, stage_v, home_v, tab_s, sem):
        cid = lax.axis_index("c")
        sid = lax.axis_index("s")
        wid = sid * 2 + cid
        idx_desc = pltpu.async_copy(
            idx_hbm.at[pl.ds(wid * 3 * _SPT, 3 * _SPT)], idx_v, sem)

        @pl.when(sid == 0)
        def _():
            pltpu.sync_copy(ctab_hbm, tab_s)

        plsc.subcore_barrier()
        idx_desc.wait()

        tiles_per_batch = _NW // _NB
        batch = wid // tiles_per_batch
        a0g = batch * _NALL + (wid % tiles_per_batch) * _APT
        for comp in range(3):
            pltpu.sync_copy(ctab_hbm.at[pl.ds(comp * _NTAB + a0g, _APT)],
                            home_v.at[pl.ds(comp * _APT, _APT)])
        pltpu.async_copy(tab_s.at[idx_v], stage_v, sem).wait()

        hregs = [[home_v[pl.ds(comp * _APT + u * 16, 16)]
                  for u in range(_APT // 16)] for comp in range(3)]

        def body(k, carry):
            for u in range(_APT // 16):
                off = k * _APT + u * 16
                for comp in range(3):
                    soff = comp * _SPT + off
                    d = stage_v[pl.ds(soff, 16)] - hregs[comp][u]
                    stage_v[pl.ds(soff, 16)] = d
            return carry

        lax.fori_loop(0, _NR, body, 0)
        pltpu.sync_copy(stage_v, out_hbm.at[pl.ds(wid * 3 * _SPT, 3 * _SPT)])

    return run(idx_chunks, ctab)


def _tc_body(atype_ref, diff_ref, tav_ref, tsd_ref,
             w0_ref, b0_ref, w1_ref, b1_ref, w2_ref, b2_ref, out_ref):
    sel0 = jnp.broadcast_to((atype_ref[0, 0, :] == 0)[None, :], (_NR, _TILE))
    d = [jnp.concatenate([diff_ref[0, comp], diff_ref[1, comp]], axis=1)
         for comp in range(3)]
    len2 = d[0] * d[0] + d[1] * d[1] + d[2] * d[2]
    dist = jnp.sqrt(len2)
    uu = (dist - _RMIN) * (1.0 / (_RMAX - _RMIN))
    vv = uu * uu * uu * (-6.0 * uu * uu + 15.0 * uu - 10.0) + 1.0
    w = jnp.where(dist >= _RMAX, 0.0, jnp.where(dist <= _RMIN, 1.0, vv))
    ll = dist * dist
    rr = []
    for comp in range(3):
        ta = jnp.where(sel0, tav_ref[0, comp], tav_ref[1, comp])
        td = jnp.where(sel0, tsd_ref[0, comp], tsd_ref[1, comp])
        rr.append(((d[comp] / ll) * w - ta) / td)
    ra = [r[0:24] for r in rr]
    rb = [r[24:_NR] for r in rr]

    acc = [jnp.zeros((_TILE,), jnp.float32) for _ in range(_NG)]
    for e, j_is_a, k_is_a, scale in _BLOCKS:
        xj = ra if j_is_a else rb
        yk = ra if k_is_a else rb
        nj, nk = xj[0].shape[0], yk[0].shape[0]
        env = None
        for comp in range(3):
            aa = jnp.broadcast_to(xj[comp][:, None, :], (nj, nk, _TILE))
            bb = jnp.broadcast_to(yk[comp][None, :, :], (nj, nk, _TILE))
            t = (aa * bb).reshape(nj * nk, _TILE)
            env = t if env is None else env + t
        h1 = [jnp.tanh(env * w0_ref[e * 2 + c] + b0_ref[e * 2 + c]) + env
              for c in range(2)]
        h2 = []
        for dd in range(4):
            z = (h1[0] * w1_ref[e * 8 + dd] + h1[1] * w1_ref[e * 8 + 4 + dd]
                 + b1_ref[e * 4 + dd])
            h2.append(jnp.tanh(z) + h1[dd % 2])
        envs = env * scale
        for mm in range(_NG):
            z = (h2[0] * w2_ref[e * 32 + mm] + h2[1] * w2_ref[e * 32 + 8 + mm]
                 + h2[2] * w2_ref[e * 32 + 16 + mm]
                 + h2[3] * w2_ref[e * 32 + 24 + mm] + b2_ref[e * 8 + mm])
            g = jnp.tanh(z) + h2[mm % 4]
            acc[mm] = acc[mm] + jnp.sum(envs * g, axis=0)
    out_ref[...] = jnp.stack(acc, axis=0)


def _tc_call(atype, diff, tav, tsd, params):
    smem = pl.BlockSpec(memory_space=pltpu.SMEM)
    return pl.pallas_call(
        _tc_body,
        grid=(_GRID,),
        in_specs=[
            pl.BlockSpec((1, 1, _TILE), lambda i: (i, 0, 0)),
            pl.BlockSpec((2, 3, _NR, _APT), lambda i: (i, 0, 0, 0)),
            pl.BlockSpec((2, 3, _NR, _TILE), lambda i: (0, 0, 0, 0)),
            pl.BlockSpec((2, 3, _NR, _TILE), lambda i: (0, 0, 0, 0)),
            smem, smem, smem, smem, smem, smem,
        ],
        out_specs=pl.BlockSpec((_NG, _TILE), lambda i: (0, i)),
        out_shape=jax.ShapeDtypeStruct((_NG, _NB * _NLOC), jnp.float32),
    )(atype, diff, tav, tsd, *params)


def _pad_cols(x, padval):
    cols = x.astype(jnp.float32)[:, :, 1:4].transpose(0, 2, 1)
    padded = jnp.concatenate(
        [cols[:, :, :20], jnp.full((2, 3, 4), padval, jnp.float32),
         cols[:, :, 20:]], axis=2)
    return jnp.broadcast_to(padded[..., None], (2, 3, _NR, _TILE))


def _prep_indices(nlist):
    off = (jnp.arange(_NB, dtype=jnp.int32) * _NALL)[:, None, None]
    idx = jnp.where(nlist >= 0, nlist + off, _SENT)
    idx = idx.reshape(_NB * _NLOC, _NNEI)
    pad = jnp.full((_NB * _NLOC, 4), _SENT, jnp.int32)
    idx = jnp.concatenate([idx[:, :20], pad, idx[:, 20:]], axis=1)
    idx = idx.reshape(_NW, _APT, _NR).transpose(0, 2, 1).reshape(_NW, 1, _SPT)
    comp_off = (jnp.arange(3, dtype=jnp.int32) * _NTAB)[None, :, None]
    return (idx + comp_off).reshape(-1)


def kernel(nlist, extended_coord, extended_atype, mean, stddev,
           W0, b0, W1, b1, W2, b2):
    nlist = nlist.astype(jnp.int32)
    coord = extended_coord.astype(jnp.float32).reshape(_NB * _NALL, 3)
    sent = jnp.array([[_FAR, 0.0, 0.0]], jnp.float32)
    planes = jnp.concatenate([coord, sent], axis=0).T
    ctab = jnp.pad(planes, ((0, 0), (0, _NTAB - _SENT - 1))).reshape(-1)
    atype = extended_atype.astype(jnp.int32)[:, :_NLOC].reshape(
        _GRID, 1, _TILE)
    idx_chunks = _prep_indices(nlist)
    diff = _sc_gather(idx_chunks, ctab)
    diff = diff.reshape(_NW, 3, _NR, _APT)
    tav = _pad_cols(mean, 0.0)
    tsd = _pad_cols(stddev, 1.0)
    params = [p.astype(jnp.float32).reshape(-1)
              for p in (W0, b0, W1, b1, W2, b2)]
    outt = _tc_call(atype, diff, tav, tsd, params)
    return outt.T.reshape(_NB, _NLOC, _NG)


# speedup vs baseline: 19.8806x; 1.0762x over previous
import functools

import jax
import jax.numpy as jnp
from jax import lax
from jax.experimental import pallas as pl
from jax.experimental.pallas import tpu as pltpu
from jax.experimental.pallas import tpu_sc as plsc

_NB, _NLOC, _NALL, _NNEI, _NG = 2, 1024, 1280, 60, 8
_RMIN, _RMAX = 0.5, 4.0
_NW = 32
_APT = (_NB * _NLOC) // _NW
_NR = 64
_SPT = _NR * _APT
_SENT = _NB * _NALL
_NTAB = _SENT + 8
_FAR = 1.0e4
_TILE = 128
_GRID = (_NB * _NLOC) // _TILE
_BLOCKS = ((0, True, True, 1.0 / 400.0),
           (2, True, False, 1.0 / 800.0),
           (3, False, False, 1.0 / 1600.0))


def _sc_gather(idx_chunks, ctab):
    mesh = plsc.VectorSubcoreMesh(core_axis_name="c", subcore_axis_name="s",
                                  num_cores=2, num_subcores=16)


    @functools.partial(
        pl.kernel,
        out_type=jax.ShapeDtypeStruct((_NW * 3 * _SPT,), jnp.float32),
        mesh=mesh,
        scratch_types=[
            pltpu.VMEM((3 * _SPT,), jnp.int32),
            pltpu.VMEM((3 * _SPT,), jnp.float32),
            pltpu.VMEM((3 * _APT,), jnp.float32),
            pltpu.VMEM_SHARED((3 * _NTAB,), jnp.float32),
            pltpu.SemaphoreType.DMA,
        ],
    )
    def run(idx_hbm, ctab_hbm, out_hbm, idx_v, stage_v, home_v, tab_s, sem):
        cid = lax.axis_index("c")
        sid = lax.axis_index("s")
        wid = sid * 2 + cid
        idx_desc = pltpu.async_copy(
            idx_hbm.at[pl.ds(wid * 3 * _SPT, 3 * _SPT)], idx_v, sem)

        @pl.when(sid == 0)
        def _():
            pltpu.sync_copy(ctab_hbm, tab_s)

        plsc.subcore_barrier()
        idx_desc.wait()

        tiles_per_batch = _NW // _NB
        batch = wid // tiles_per_batch
        a0g = batch * _NALL + (wid % tiles_per_batch) * _APT
        for comp in range(3):
            pltpu.sync_copy(ctab_hbm.at[pl.ds(comp * _NTAB + a0g, _APT)],
                            home_v.at[pl.ds(comp * _APT, _APT)])
        pltpu.async_copy(tab_s.at[idx_v], stage_v, sem).wait()

        hregs = [[home_v[pl.ds(comp * _APT + u * 16, 16)]
                  for u in range(_APT // 16)] for comp in range(3)]

        def body(k, carry):
            for u in range(_APT // 16):
                off = k * _APT + u * 16
                for comp in range(3):
                    soff = comp * _SPT + off
                    d = stage_v[pl.ds(soff, 16)] - hregs[comp][u]
                    stage_v[pl.ds(soff, 16)] = d
            return carry

        lax.fori_loop(0, _NR, body, 0)
        pltpu.sync_copy(stage_v, out_hbm.at[pl.ds(wid * 3 * _SPT, 3 * _SPT)])

    return run(idx_chunks, ctab)


def _tc_body(atype_ref, diff_ref, tav_ref, tsd_ref,
             w0_ref, b0_ref, w1_ref, b1_ref, w2_ref, b2_ref, out_ref):
    sel0 = jnp.broadcast_to((atype_ref[0, 0, :] == 0)[None, :], (_NR, _TILE))
    d = [jnp.concatenate([diff_ref[0, comp], diff_ref[1, comp]], axis=1)
         for comp in range(3)]
    len2 = d[0] * d[0] + d[1] * d[1] + d[2] * d[2]
    dist = jnp.sqrt(len2)
    uu = (dist - _RMIN) * (1.0 / (_RMAX - _RMIN))
    vv = uu * uu * uu * (-6.0 * uu * uu + 15.0 * uu - 10.0) + 1.0
    w = jnp.where(dist >= _RMAX, 0.0, jnp.where(dist <= _RMIN, 1.0, vv))
    ll = dist * dist
    rr = []
    for comp in range(3):
        ta = jnp.where(sel0, tav_ref[0, comp], tav_ref[1, comp])
        td = jnp.where(sel0, tsd_ref[0, comp], tsd_ref[1, comp])
        rr.append(((d[comp] / ll) * w - ta) / td)
    ra = [r[0:24] for r in rr]
    rj = [r[0:20] for r in rr]
    rb = [r[24:_NR] for r in rr]

    acc = [jnp.zeros((_TILE,), jnp.float32) for _ in range(_NG)]
    for e, j_is_a, k_is_a, scale in _BLOCKS:
        xj = rj if j_is_a else rb
        yk = ra if k_is_a else rb
        nj, nk = xj[0].shape[0], yk[0].shape[0]
        env = None
        for comp in range(3):
            aa = jnp.broadcast_to(xj[comp][:, None, :], (nj, nk, _TILE))
            bb = jnp.broadcast_to(yk[comp][None, :, :], (nj, nk, _TILE))
            t = (aa * bb).reshape(nj * nk, _TILE)
            env = t if env is None else env + t
        h1 = [jnp.tanh(env * w0_ref[e * 2 + c] + b0_ref[e * 2 + c]) + env
              for c in range(2)]
        h2 = []
        for dd in range(4):
            z = (h1[0] * w1_ref[e * 8 + dd] + h1[1] * w1_ref[e * 8 + 4 + dd]
                 + b1_ref[e * 4 + dd])
            h2.append(jnp.tanh(z) + h1[dd % 2])
        envs = env * scale
        for mm in range(_NG):
            z = (h2[0] * w2_ref[e * 32 + mm] + h2[1] * w2_ref[e * 32 + 8 + mm]
                 + h2[2] * w2_ref[e * 32 + 16 + mm]
                 + h2[3] * w2_ref[e * 32 + 24 + mm] + b2_ref[e * 8 + mm])
            g = jnp.tanh(z) + h2[mm % 4]
            acc[mm] = acc[mm] + jnp.sum(envs * g, axis=0)
    out_ref[...] = jnp.stack(acc, axis=0)


def _tc_call(atype, diff, tav, tsd, params):
    smem = pl.BlockSpec(memory_space=pltpu.SMEM)
    return pl.pallas_call(
        _tc_body,
        grid=(_GRID,),
        in_specs=[
            pl.BlockSpec((1, 1, _TILE), lambda i: (i, 0, 0)),
            pl.BlockSpec((2, 3, _NR, _APT), lambda i: (i, 0, 0, 0)),
            pl.BlockSpec((2, 3, _NR, _TILE), lambda i: (0, 0, 0, 0)),
            pl.BlockSpec((2, 3, _NR, _TILE), lambda i: (0, 0, 0, 0)),
            smem, smem, smem, smem, smem, smem,
        ],
        out_specs=pl.BlockSpec((_NG, _TILE), lambda i: (0, i)),
        out_shape=jax.ShapeDtypeStruct((_NG, _NB * _NLOC), jnp.float32),
    )(atype, diff, tav, tsd, *params)


def _pad_cols(x, padval):
    cols = x.astype(jnp.float32)[:, :, 1:4].transpose(0, 2, 1)
    padded = jnp.concatenate(
        [cols[:, :, :20], jnp.full((2, 3, 4), padval, jnp.float32),
         cols[:, :, 20:]], axis=2)
    return jnp.broadcast_to(padded[..., None], (2, 3, _NR, _TILE))


def _prep_indices(nlist):
    off = (jnp.arange(_NB, dtype=jnp.int32) * _NALL)[:, None, None]
    idx = jnp.where(nlist >= 0, nlist + off, _SENT)
    idx = idx.reshape(_NB * _NLOC, _NNEI)
    pad = jnp.full((_NB * _NLOC, 4), _SENT, jnp.int32)
    idx = jnp.concatenate([idx[:, :20], pad, idx[:, 20:]], axis=1)
    idx = idx.reshape(_NW, _APT, _NR).transpose(0, 2, 1).reshape(_NW, 1, _SPT)
    comp_off = (jnp.arange(3, dtype=jnp.int32) * _NTAB)[None, :, None]
    return (idx + comp_off).reshape(-1)


def kernel(nlist, extended_coord, extended_atype, mean, stddev,
           W0, b0, W1, b1, W2, b2):
    nlist = nlist.astype(jnp.int32)
    coord = extended_coord.astype(jnp.float32).reshape(_NB * _NALL, 3)
    sent = jnp.array([[_FAR, 0.0, 0.0]], jnp.float32)
    planes = jnp.concatenate([coord, sent], axis=0).T
    ctab = jnp.pad(planes, ((0, 0), (0, _NTAB - _SENT - 1))).reshape(-1)
    atype = extended_atype.astype(jnp.int32)[:, :_NLOC].reshape(
        _GRID, 1, _TILE)
    idx_chunks = _prep_indices(nlist)
    diff = _sc_gather(idx_chunks, ctab)
    diff = diff.reshape(_NW, 3, _NR, _APT)
    tav = _pad_cols(mean, 0.0)
    tsd = _pad_cols(stddev, 1.0)
    params = [p.astype(jnp.float32).reshape(-1)
              for p in (W0, b0, W1, b1, W2, b2)]
    outt = _tc_call(atype, diff, tav, tsd, params)
    return outt.T.reshape(_NB, _NLOC, _NG)


# speedup vs baseline: 29.7322x; 1.4955x over previous
import functools

import jax
import jax.numpy as jnp
from jax import lax
from jax.experimental import pallas as pl
from jax.experimental.pallas import tpu as pltpu
from jax.experimental.pallas import tpu_sc as plsc

_NB, _NLOC, _NALL, _NNEI, _NG = 2, 1024, 1280, 60, 8
_RMIN, _RMAX = 0.5, 4.0
_NW = 32
_APT = (_NB * _NLOC) // _NW
_NR = 64
_SPT = _NR * _APT
_SENT = _NB * _NALL
_NTAB = _SENT + 8
_FAR = 1.0e4
_TILE = 128
_GRID = (_NB * _NLOC) // _TILE
_BLOCKS = ((0, True, True, 1.0 / 400.0),
           (2, True, False, 1.0 / 800.0),
           (3, False, False, 1.0 / 1600.0))


def _sc_gather(idx_chunks, ctab):
    mesh = plsc.VectorSubcoreMesh(core_axis_name="c", subcore_axis_name="s",
                                  num_cores=2, num_subcores=16)


    @functools.partial(
        pl.kernel,
        out_type=jax.ShapeDtypeStruct((_NW * 3 * _SPT,), jnp.float32),
        mesh=mesh,
        scratch_types=[
            pltpu.VMEM((3 * _SPT,), jnp.int32),
            pltpu.VMEM((3 * _SPT,), jnp.float32),
            pltpu.VMEM((3 * _APT,), jnp.float32),
            pltpu.VMEM_SHARED((3 * _NTAB,), jnp.float32),
            pltpu.SemaphoreType.DMA,
        ],
    )
    def run(idx_hbm, ctab_hbm, out_hbm, idx_v, stage_v, home_v, tab_s, sem):
        cid = lax.axis_index("c")
        sid = lax.axis_index("s")
        wid = sid * 2 + cid
        idx_desc = pltpu.async_copy(
            idx_hbm.at[pl.ds(wid * 3 * _SPT, 3 * _SPT)], idx_v, sem)

        @pl.when(sid == 0)
        def _():
            pltpu.sync_copy(ctab_hbm, tab_s)

        plsc.subcore_barrier()
        idx_desc.wait()

        tiles_per_batch = _NW // _NB
        batch = wid // tiles_per_batch
        a0g = batch * _NALL + (wid % tiles_per_batch) * _APT
        for comp in range(3):
            pltpu.sync_copy(ctab_hbm.at[pl.ds(comp * _NTAB + a0g, _APT)],
                            home_v.at[pl.ds(comp * _APT, _APT)])
        pltpu.async_copy(tab_s.at[idx_v], stage_v, sem).wait()

        hregs = [[home_v[pl.ds(comp * _APT + u * 16, 16)]
                  for u in range(_APT // 16)] for comp in range(3)]

        def body(k, carry):
            for u in range(_APT // 16):
                off = k * _APT + u * 16
                for comp in range(3):
                    soff = comp * _SPT + off
                    d = stage_v[pl.ds(soff, 16)] - hregs[comp][u]
                    stage_v[pl.ds(soff, 16)] = d
            return carry

        lax.fori_loop(0, _NR, body, 0)
        pltpu.sync_copy(stage_v, out_hbm.at[pl.ds(wid * 3 * _SPT, 3 * _SPT)])

    return run(idx_chunks, ctab)


def _tc_body(atype_ref, diff_ref, tav_ref, tsd_ref,
             w0_ref, b0_ref, w1_ref, b1_ref, w2_ref, b2_ref, out_ref):
    sel0 = jnp.broadcast_to((atype_ref[0, 0, :] == 0)[None, :], (_NR, _TILE))
    d = [jnp.concatenate([diff_ref[0, comp], diff_ref[1, comp]], axis=1)
         for comp in range(3)]
    len2 = d[0] * d[0] + d[1] * d[1] + d[2] * d[2]
    dist = jnp.sqrt(len2)
    uu = (dist - _RMIN) * (1.0 / (_RMAX - _RMIN))
    vv = uu * uu * uu * (-6.0 * uu * uu + 15.0 * uu - 10.0) + 1.0
    w = jnp.where(dist >= _RMAX, 0.0, jnp.where(dist <= _RMIN, 1.0, vv))
    ll = dist * dist
    rr = []
    for comp in range(3):
        ta = jnp.where(sel0, tav_ref[0, comp], tav_ref[1, comp])
        td = jnp.where(sel0, tsd_ref[0, comp], tsd_ref[1, comp])
        rr.append(((d[comp] / ll) * w - ta) / td)
    ra = [r[0:24] for r in rr]
    rj = [r[0:20] for r in rr]
    rb = [r[24:_NR] for r in rr]

    acc8 = [jnp.zeros((8, _TILE), jnp.float32) for _ in range(_NG)]
    for e, j_is_a, k_is_a, scale in _BLOCKS:
        xj = rj if j_is_a else rb
        yk = ra if k_is_a else rb
        nj, nk = xj[0].shape[0], yk[0].shape[0]
        w0 = [w0_ref[e * 2 + c] for c in range(2)]
        b0 = [b0_ref[e * 2 + c] for c in range(2)]
        w1 = [[w1_ref[e * 8 + ci * 4 + dd] for dd in range(4)]
              for ci in range(2)]
        b1 = [b1_ref[e * 4 + dd] for dd in range(4)]
        w2 = [[w2_ref[e * 32 + ci * 8 + mm] for mm in range(_NG)]
              for ci in range(4)]
        b2 = [b2_ref[e * 8 + mm] for mm in range(_NG)]
        for j in range(nj):
            aa = [jnp.broadcast_to(xj[comp][j:j + 1, :], (8, _TILE))
                  for comp in range(3)]
            for kc in range(nk // 8):
                env = None
                for comp in range(3):
                    t = aa[comp] * yk[comp][kc * 8:(kc + 1) * 8, :]
                    env = t if env is None else env + t
                h1 = [jnp.tanh(env * w0[c] + b0[c]) + env for c in range(2)]
                h2 = [jnp.tanh(h1[0] * w1[0][dd] + h1[1] * w1[1][dd]
                               + b1[dd]) + h1[dd % 2] for dd in range(4)]
                envs = env * scale
                for mm in range(_NG):
                    z = (h2[0] * w2[0][mm] + h2[1] * w2[1][mm]
                         + h2[2] * w2[2][mm] + h2[3] * w2[3][mm] + b2[mm])
                    g = jnp.tanh(z) + h2[mm % 4]
                    acc8[mm] = acc8[mm] + envs * g
    out_ref[...] = jnp.stack([jnp.sum(a, axis=0) for a in acc8], axis=0)


def _tc_call(atype, diff, tav, tsd, params):
    smem = pl.BlockSpec(memory_space=pltpu.SMEM)
    return pl.pallas_call(
        _tc_body,
        grid=(_GRID,),
        in_specs=[
            pl.BlockSpec((1, 1, _TILE), lambda i: (i, 0, 0)),
            pl.BlockSpec((2, 3, _NR, _APT), lambda i: (i, 0, 0, 0)),
            pl.BlockSpec((2, 3, _NR, _TILE), lambda i: (0, 0, 0, 0)),
            pl.BlockSpec((2, 3, _NR, _TILE), lambda i: (0, 0, 0, 0)),
            smem, smem, smem, smem, smem, smem,
        ],
        out_specs=pl.BlockSpec((_NG, _TILE), lambda i: (0, i)),
        out_shape=jax.ShapeDtypeStruct((_NG, _NB * _NLOC), jnp.float32),
    )(atype, diff, tav, tsd, *params)


def _pad_cols(x, padval):
    cols = x.astype(jnp.float32)[:, :, 1:4].transpose(0, 2, 1)
    padded = jnp.concatenate(
        [cols[:, :, :20], jnp.full((2, 3, 4), padval, jnp.float32),
         cols[:, :, 20:]], axis=2)
    return jnp.broadcast_to(padded[..., None], (2, 3, _NR, _TILE))


def _prep_indices(nlist):
    off = (jnp.arange(_NB, dtype=jnp.int32) * _NALL)[:, None, None]
    idx = jnp.where(nlist >= 0, nlist + off, _SENT)
    idx = idx.reshape(_NB * _NLOC, _NNEI)
    pad = jnp.full((_NB * _NLOC, 4), _SENT, jnp.int32)
    idx = jnp.concatenate([idx[:, :20], pad, idx[:, 20:]], axis=1)
    idx = idx.reshape(_NW, _APT, _NR).transpose(0, 2, 1).reshape(_NW, 1, _SPT)
    comp_off = (jnp.arange(3, dtype=jnp.int32) * _NTAB)[None, :, None]
    return (idx + comp_off).reshape(-1)


def kernel(nlist, extended_coord, extended_atype, mean, stddev,
           W0, b0, W1, b1, W2, b2):
    nlist = nlist.astype(jnp.int32)
    coord = extended_coord.astype(jnp.float32).reshape(_NB * _NALL, 3)
    sent = jnp.array([[_FAR, 0.0, 0.0]], jnp.float32)
    planes = jnp.concatenate([coord, sent], axis=0).T
    ctab = jnp.pad(planes, ((0, 0), (0, _NTAB - _SENT - 1))).reshape(-1)
    atype = extended_atype.astype(jnp.int32)[:, :_NLOC].reshape(
        _GRID, 1, _TILE)
    idx_chunks = _prep_indices(nlist)
    diff = _sc_gather(idx_chunks, ctab)
    diff = diff.reshape(_NW, 3, _NR, _APT)
    tav = _pad_cols(mean, 0.0)
    tsd = _pad_cols(stddev, 1.0)
    params = [p.astype(jnp.float32).reshape(-1)
              for p in (W0, b0, W1, b1, W2, b2)]
    outt = _tc_call(atype, diff, tav, tsd, params)
    return outt.T.reshape(_NB, _NLOC, _NG)
